# baseline (speedup 1.0000x reference)
import functools

import jax
import jax.numpy as jnp
from jax import lax
from jax.experimental import pallas as pl
from jax.experimental.pallas import tpu as pltpu
from jax.experimental.pallas import tpu_sc as plsc

NITEM = 1000000
EMB_DIM = 64
BATCH = 16384

NC = 2
NS = 16
LANES = 16
NW = NC * NS
BPW = BATCH // NW
CHUNK = 128
NCHUNK = BPW // CHUNK
GROUPS = BPW // LANES


def _sc_body(center_hbm, context_hbm, table1_hbm, table2_hbm, out_hbm,
             cidx_v, xidx_v, rows1_v, rows2_v, out_v, sem):
    wid = lax.axis_index("s") * NC + lax.axis_index("c")
    base = wid * BPW

    for j in range(NCHUNK):
        pltpu.sync_copy(center_hbm.at[pl.ds(base + j * CHUNK, CHUNK)],
                        cidx_v.at[j])
        pltpu.sync_copy(context_hbm.at[pl.ds(base + j * CHUNK, CHUNK)],
                        xidx_v.at[j])

    copies = []
    for j in range(NCHUNK):
        copies.append(pltpu.async_copy(
            table1_hbm.at[cidx_v.at[j]], rows1_v.at[j], sem))
        copies.append(pltpu.async_copy(
            table2_hbm.at[xidx_v.at[j]], rows2_v.at[j], sem))
    for cp in copies:
        cp.wait()

    iota = lax.iota(jnp.int32, LANES)
    lane_masks = [iota == r for r in range(LANES)]

    for j in range(NCHUNK):
        def grp_body(g, carry, j=j):
            acc = jnp.zeros((LANES,), jnp.float32)
            for r in range(LANES):
                c = g * LANES + r
                s = jnp.zeros((LANES,), jnp.float32)
                for k in range(EMB_DIM // LANES):
                    a = rows1_v[j, c, pl.ds(k * LANES, LANES)]
                    b = rows2_v[j, c, pl.ds(k * LANES, LANES)]
                    s = s + a * b
                acc = jnp.where(lane_masks[r], jnp.sum(s), acc)
            y = 1.0 / (1.0 + jnp.exp(-acc))
            out_v[pl.ds(j * CHUNK + g * LANES, LANES)] = y
            return carry

        lax.fori_loop(0, CHUNK // LANES, grp_body, 0)

    pltpu.sync_copy(out_v, out_hbm.at[pl.ds(base, BPW)])


def kernel(center, context, table1, table2):
    mesh = plsc.VectorSubcoreMesh(core_axis_name="c", subcore_axis_name="s",
                                  num_cores=NC, num_subcores=NS)
    run = pl.kernel(
        _sc_body,
        out_type=jax.ShapeDtypeStruct((BATCH,), jnp.float32),
        mesh=mesh,
        compiler_params=pltpu.CompilerParams(needs_layout_passes=False,
                                             use_tc_tiling_on_sc=False),
        scratch_types=[
            pltpu.VMEM((NCHUNK, CHUNK), jnp.int32),
            pltpu.VMEM((NCHUNK, CHUNK), jnp.int32),
            pltpu.VMEM((NCHUNK, CHUNK, EMB_DIM), jnp.float32),
            pltpu.VMEM((NCHUNK, CHUNK, EMB_DIM), jnp.float32),
            pltpu.VMEM((BPW,), jnp.float32),
            pltpu.SemaphoreType.DMA,
        ],
    )
    return run(center.astype(jnp.int32), context.astype(jnp.int32),
               table1, table2)


# speedup vs baseline: 1.5656x; 1.5656x over previous
import functools

import jax
import jax.numpy as jnp
from jax import lax
from jax.experimental import pallas as pl
from jax.experimental.pallas import tpu as pltpu
from jax.experimental.pallas import tpu_sc as plsc

NITEM = 1000000
EMB_DIM = 64
BATCH = 16384

NC = 2
NS = 16
LANES = 16
NW = NC * NS
BPW = BATCH // NW
CHUNK = 128
NCHUNK = BPW // CHUNK


def _sc_body(center_hbm, context_hbm, table1_hbm, table2_hbm, out_hbm,
             cidx_v, xidx_v, rows1_v, rows2_v, out_v, sem):
    wid = lax.axis_index("s") * NC + lax.axis_index("c")
    base = wid * BPW

    for j in range(NCHUNK):
        pltpu.sync_copy(center_hbm.at[pl.ds(base + j * CHUNK, CHUNK)],
                        cidx_v.at[j])
        pltpu.sync_copy(context_hbm.at[pl.ds(base + j * CHUNK, CHUNK)],
                        xidx_v.at[j])

    iota = lax.iota(jnp.int32, LANES)
    lane_masks = [iota == r for r in range(LANES)]

    for j in range(NCHUNK):
        def fetch_body(g, carry, j=j):
            v1 = cidx_v[j, pl.ds(g * LANES, LANES)]
            v2 = xidx_v[j, pl.ds(g * LANES, LANES)]
            for r in range(LANES):
                i = g * LANES + r
                pltpu.async_copy(table1_hbm.at[pl.ds(v1[r], 1), :],
                                 rows1_v.at[pl.ds(i, 1), :], sem)
                pltpu.async_copy(table2_hbm.at[pl.ds(v2[r], 1), :],
                                 rows2_v.at[pl.ds(i, 1), :], sem)
            return carry

        lax.fori_loop(0, CHUNK // LANES, fetch_body, 0)

        def drain_body(i, carry):
            pltpu.make_async_copy(table1_hbm.at[pl.ds(0, 1), :],
                                  rows1_v.at[pl.ds(i, 1), :], sem).wait()
            pltpu.make_async_copy(table2_hbm.at[pl.ds(0, 1), :],
                                  rows2_v.at[pl.ds(i, 1), :], sem).wait()
            return carry

        lax.fori_loop(0, CHUNK, drain_body, 0)

        def grp_body(g, carry, j=j):
            acc = jnp.zeros((LANES,), jnp.float32)
            for r in range(LANES):
                c = g * LANES + r
                s = jnp.zeros((LANES,), jnp.float32)
                for k in range(EMB_DIM // LANES):
                    a = rows1_v[c, pl.ds(k * LANES, LANES)]
                    b = rows2_v[c, pl.ds(k * LANES, LANES)]
                    s = s + a * b
                acc = jnp.where(lane_masks[r], jnp.sum(s), acc)
            y = 1.0 / (1.0 + jnp.exp(-acc))
            out_v[pl.ds(j * CHUNK + g * LANES, LANES)] = y
            return carry

        lax.fori_loop(0, CHUNK // LANES, grp_body, 0)

    pltpu.sync_copy(out_v, out_hbm.at[pl.ds(base, BPW)])


def kernel(center, context, table1, table2):
    mesh = plsc.VectorSubcoreMesh(core_axis_name="c", subcore_axis_name="s",
                                  num_cores=NC, num_subcores=NS)
    run = pl.kernel(
        _sc_body,
        out_type=jax.ShapeDtypeStruct((BATCH,), jnp.float32),
        mesh=mesh,
        compiler_params=pltpu.CompilerParams(needs_layout_passes=False),
        scratch_types=[
            pltpu.VMEM((NCHUNK, CHUNK), jnp.int32),
            pltpu.VMEM((NCHUNK, CHUNK), jnp.int32),
            pltpu.VMEM((CHUNK, EMB_DIM), jnp.float32),
            pltpu.VMEM((CHUNK, EMB_DIM), jnp.float32),
            pltpu.VMEM((BPW,), jnp.float32),
            pltpu.SemaphoreType.DMA,
        ],
    )
    return run(center.astype(jnp.int32), context.astype(jnp.int32),
               table1, table2)


# speedup vs baseline: 1.5714x; 1.0037x over previous
import functools

import jax
import jax.numpy as jnp
from jax import lax
from jax.experimental import pallas as pl
from jax.experimental.pallas import tpu as pltpu
from jax.experimental.pallas import tpu_sc as plsc

NITEM = 1000000
EMB_DIM = 64
BATCH = 16384

NC = 2
NS = 16
LANES = 16
NW = NC * NS
BPW = BATCH // NW
CHUNK = 128
NCHUNK = BPW // CHUNK


def _sc_body(center_hbm, context_hbm, table1_hbm, table2_hbm, out_hbm,
             cidx_v, xidx_v, rows1_v, rows2_v, out_v, sem):
    wid = lax.axis_index("s") * NC + lax.axis_index("c")
    base = wid * BPW

    for j in range(NCHUNK):
        pltpu.sync_copy(center_hbm.at[pl.ds(base + j * CHUNK, CHUNK)],
                        cidx_v.at[j])
        pltpu.sync_copy(context_hbm.at[pl.ds(base + j * CHUNK, CHUNK)],
                        xidx_v.at[j])

    iota = lax.iota(jnp.int32, LANES)
    lane_masks = [iota == r for r in range(LANES)]

    for j in range(NCHUNK):
        def fetch_body(g, carry, j=j):
            v1 = cidx_v[j, pl.ds(g * LANES, LANES)]
            v2 = xidx_v[j, pl.ds(g * LANES, LANES)]
            for r in range(LANES):
                i = g * LANES + r
                pltpu.async_copy(table1_hbm.at[pl.ds(v1[r], 1), :],
                                 rows1_v.at[pl.ds(i, 1), :], sem)
                pltpu.async_copy(table2_hbm.at[pl.ds(v2[r], 1), :],
                                 rows2_v.at[pl.ds(i, 1), :], sem)
            return carry

        lax.fori_loop(0, CHUNK // LANES, fetch_body, 0)

        def drain_body(i, carry):
            pltpu.make_async_copy(table1_hbm.at[pl.ds(0, 1), :],
                                  rows1_v.at[pl.ds(i, 1), :], sem).wait()
            pltpu.make_async_copy(table2_hbm.at[pl.ds(0, 1), :],
                                  rows2_v.at[pl.ds(i, 1), :], sem).wait()
            return carry

        lax.fori_loop(0, CHUNK, drain_body, 0)

        def grp_body(g, carry, j=j):
            acc = jnp.zeros((LANES,), jnp.float32)
            for r in range(LANES):
                c = g * LANES + r
                s = jnp.zeros((LANES,), jnp.float32)
                for k in range(EMB_DIM // LANES):
                    a = rows1_v[c, pl.ds(k * LANES, LANES)]
                    b = rows2_v[c, pl.ds(k * LANES, LANES)]
                    s = s + a * b
                acc = jnp.where(lane_masks[r], jnp.sum(s), acc)
            y = 1.0 / (1.0 + jnp.exp(-acc))
            out_v[pl.ds(j * CHUNK + g * LANES, LANES)] = y
            return carry

        lax.fori_loop(0, CHUNK // LANES, grp_body, 0)

    pltpu.sync_copy(out_v, out_hbm.at[pl.ds(base, BPW)])


def kernel(center, context, table1, table2):
    mesh = plsc.VectorSubcoreMesh(core_axis_name="c", subcore_axis_name="s",
                                  num_cores=NC, num_subcores=NS)
    run = pl.kernel(
        _sc_body,
        out_type=jax.ShapeDtypeStruct((BATCH,), jnp.float32),
        mesh=mesh,
        compiler_params=pltpu.CompilerParams(needs_layout_passes=False,
                                             use_tc_tiling_on_sc=True),
        scratch_types=[
            pltpu.VMEM((NCHUNK, CHUNK), jnp.int32),
            pltpu.VMEM((NCHUNK, CHUNK), jnp.int32),
            pltpu.VMEM((CHUNK, EMB_DIM), jnp.float32),
            pltpu.VMEM((CHUNK, EMB_DIM), jnp.float32),
            pltpu.VMEM((BPW,), jnp.float32),
            pltpu.SemaphoreType.DMA,
        ],
    )
    return run(center.astype(jnp.int32), context.astype(jnp.int32),
               table1, table2)


# speedup vs baseline: 2.1335x; 1.3577x over previous
import functools

import jax
import jax.numpy as jnp
from jax import lax
from jax.experimental import pallas as pl
from jax.experimental.pallas import tpu as pltpu
from jax.experimental.pallas import tpu_sc as plsc

NITEM = 1000000
EMB_DIM = 64
BATCH = 16384

NC = 2
NS = 16
LANES = 16
NW = NC * NS
BPW = BATCH // NW
BLK = 128
ICHUNK = 128
NICHUNK = BPW // ICHUNK


def _fetch(t1_hbm, t2_hbm, idx1, idx2, b1, b2, sem):
    base1 = pl.multiple_of((idx1 >> 7) * BLK, BLK)
    base2 = pl.multiple_of((idx2 >> 7) * BLK, BLK)
    pltpu.async_copy(t1_hbm.at[:, pl.ds(base1, BLK)], b1, sem)
    pltpu.async_copy(t2_hbm.at[:, pl.ds(base2, BLK)], b2, sem)


def _sc_body(center_hbm, context_hbm, t1_hbm, t2_hbm, out_hbm,
             cidx_v, xidx_v, c_s, x_s, blkA1, blkA2, blkB1, blkB2,
             out_v, semA, semB):
    wid = lax.axis_index("s") * NC + lax.axis_index("c")
    base = wid * BPW

    for j in range(NICHUNK):
        pltpu.sync_copy(center_hbm.at[pl.ds(base + j * ICHUNK, ICHUNK)],
                        cidx_v.at[j])
        pltpu.sync_copy(context_hbm.at[pl.ds(base + j * ICHUNK, ICHUNK)],
                        xidx_v.at[j])

    def spill_body(g, carry):
        j = g // (ICHUNK // LANES)
        gg = g % (ICHUNK // LANES)
        v1 = cidx_v[j, pl.ds(gg * LANES, LANES)]
        v2 = xidx_v[j, pl.ds(gg * LANES, LANES)]
        for r in range(LANES):
            c_s[g * LANES + r] = v1[r]
            x_s[g * LANES + r] = v2[r]
        return carry

    lax.fori_loop(0, BPW // LANES, spill_body, 0)

    iota = lax.iota(jnp.int32, LANES)
    c_vecs = [kc * LANES + iota for kc in range(EMB_DIM // LANES)]

    _fetch(t1_hbm, t2_hbm, c_s[0], x_s[0], blkA1, blkA2, semA)

    def _dot(b1, b2, col1, col2, acc, lane):
        col1v = jnp.full((LANES,), col1, jnp.int32)
        col2v = jnp.full((LANES,), col2, jnp.int32)
        s = jnp.zeros((LANES,), jnp.float32)
        for kc in range(EMB_DIM // LANES):
            a = plsc.load_gather(b1, [c_vecs[kc], col1v])
            b = plsc.load_gather(b2, [c_vecs[kc], col2v])
            s = s + a * b
        return jnp.where(lane, jnp.sum(s), acc)

    def pair_body(g, acc):
        k0 = 2 * g
        k1 = k0 + 1
        _fetch(t1_hbm, t2_hbm, c_s[k1], x_s[k1], blkB1, blkB2, semB)
        pltpu.make_async_copy(t1_hbm.at[:, pl.ds(0, BLK)], blkA1, semA).wait()
        pltpu.make_async_copy(t1_hbm.at[:, pl.ds(0, BLK)], blkA2, semA).wait()
        acc = _dot(blkA1, blkA2, c_s[k0] & (BLK - 1), x_s[k0] & (BLK - 1),
                   acc, iota == (k0 % LANES))

        @pl.when(g + 1 < BPW // 2)
        def _():
            _fetch(t1_hbm, t2_hbm, c_s[k0 + 2], x_s[k0 + 2],
                   blkA1, blkA2, semA)

        pltpu.make_async_copy(t1_hbm.at[:, pl.ds(0, BLK)], blkB1, semB).wait()
        pltpu.make_async_copy(t1_hbm.at[:, pl.ds(0, BLK)], blkB2, semB).wait()
        acc = _dot(blkB1, blkB2, c_s[k1] & (BLK - 1), x_s[k1] & (BLK - 1),
                   acc, iota == (k1 % LANES))

        @pl.when(k1 % LANES == LANES - 1)
        def _():
            y = 1.0 / (1.0 + jnp.exp(-acc))
            out_v[pl.ds((k1 // LANES) * LANES, LANES)] = y

        return jnp.where(k1 % LANES == LANES - 1,
                         jnp.zeros((LANES,), jnp.float32), acc)

    lax.fori_loop(0, BPW // 2, pair_body, jnp.zeros((LANES,), jnp.float32))

    pltpu.sync_copy(out_v, out_hbm.at[pl.ds(base, BPW)])


def kernel(center, context, table1, table2):
    mesh = plsc.VectorSubcoreMesh(core_axis_name="c", subcore_axis_name="s",
                                  num_cores=NC, num_subcores=NS)
    run = pl.kernel(
        _sc_body,
        out_type=jax.ShapeDtypeStruct((BATCH,), jnp.float32),
        mesh=mesh,
        compiler_params=pltpu.CompilerParams(needs_layout_passes=False,
                                             use_tc_tiling_on_sc=True),
        scratch_types=[
            pltpu.VMEM((NICHUNK, ICHUNK), jnp.int32),
            pltpu.VMEM((NICHUNK, ICHUNK), jnp.int32),
            pltpu.SMEM((BPW,), jnp.int32),
            pltpu.SMEM((BPW,), jnp.int32),
            pltpu.VMEM((EMB_DIM, BLK), jnp.float32),
            pltpu.VMEM((EMB_DIM, BLK), jnp.float32),
            pltpu.VMEM((EMB_DIM, BLK), jnp.float32),
            pltpu.VMEM((EMB_DIM, BLK), jnp.float32),
            pltpu.VMEM((BPW,), jnp.float32),
            pltpu.SemaphoreType.DMA,
            pltpu.SemaphoreType.DMA,
        ],
    )
    return run(center.astype(jnp.int32), context.astype(jnp.int32),
               jnp.swapaxes(table1, 0, 1), jnp.swapaxes(table2, 0, 1))


# speedup vs baseline: 2.6313x; 1.2333x over previous
import functools

import jax
import jax.numpy as jnp
from jax import lax
from jax.experimental import pallas as pl
from jax.experimental.pallas import tpu as pltpu
from jax.experimental.pallas import tpu_sc as plsc

NITEM = 1000000
EMB_DIM = 64
BATCH = 16384

NC = 2
NS = 16
LANES = 16
NW = NC * NS
BPW = BATCH // NW
BLK = 128
ICHUNK = 128
NICHUNK = BPW // ICHUNK


def _fetch(t1_hbm, t2_hbm, idx1, idx2, b1, b2, sem):
    base1 = pl.multiple_of((idx1 >> 7) * BLK, BLK)
    base2 = pl.multiple_of((idx2 >> 7) * BLK, BLK)
    pltpu.async_copy(t1_hbm.at[:, pl.ds(base1, BLK)], b1, sem)
    pltpu.async_copy(t2_hbm.at[:, pl.ds(base2, BLK)], b2, sem)


def _sc_body(center_hbm, context_hbm, t1_hbm, t2_hbm, out_hbm,
             cidx_v, xidx_v, c_s, x_s,
             blk0_1, blk0_2, blk1_1, blk1_2, blk2_1, blk2_2, blk3_1, blk3_2,
             out_v, sem0, sem1, sem2, sem3):
    wid = lax.axis_index("s") * NC + lax.axis_index("c")
    base = wid * BPW

    for j in range(NICHUNK):
        pltpu.sync_copy(center_hbm.at[pl.ds(base + j * ICHUNK, ICHUNK)],
                        cidx_v.at[j])
        pltpu.sync_copy(context_hbm.at[pl.ds(base + j * ICHUNK, ICHUNK)],
                        xidx_v.at[j])

    def spill_body(g, carry):
        j = g // (ICHUNK // LANES)
        gg = g % (ICHUNK // LANES)
        v1 = cidx_v[j, pl.ds(gg * LANES, LANES)]
        v2 = xidx_v[j, pl.ds(gg * LANES, LANES)]
        for r in range(LANES):
            c_s[g * LANES + r] = v1[r]
            x_s[g * LANES + r] = v2[r]
        return carry

    lax.fori_loop(0, BPW // LANES, spill_body, 0)

    iota = lax.iota(jnp.int32, LANES)
    c_vecs = [kc * LANES + iota for kc in range(EMB_DIM // LANES)]

    slots = [(blk0_1, blk0_2, sem0), (blk1_1, blk1_2, sem1),
             (blk2_1, blk2_2, sem2), (blk3_1, blk3_2, sem3)]
    NSLOT = len(slots)

    for s, (b1, b2, sem) in enumerate(slots):
        _fetch(t1_hbm, t2_hbm, c_s[s], x_s[s], b1, b2, sem)

    def _dot(b1, b2, col1, col2, acc, lane):
        col1v = jnp.full((LANES,), col1, jnp.int32)
        col2v = jnp.full((LANES,), col2, jnp.int32)
        s = jnp.zeros((LANES,), jnp.float32)
        for kc in range(EMB_DIM // LANES):
            a = plsc.load_gather(b1, [c_vecs[kc], col1v])
            b = plsc.load_gather(b2, [c_vecs[kc], col2v])
            s = s + a * b
        return jnp.where(lane, jnp.sum(s), acc)

    def ring_body(g, acc):
        for s, (b1, b2, sem) in enumerate(slots):
            k = NSLOT * g + s
            pltpu.make_async_copy(t1_hbm.at[:, pl.ds(0, BLK)], b1, sem).wait()
            pltpu.make_async_copy(t1_hbm.at[:, pl.ds(0, BLK)], b2, sem).wait()
            acc = _dot(b1, b2, c_s[k] & (BLK - 1), x_s[k] & (BLK - 1),
                       acc, iota == (k % LANES))

            @pl.when(g + 1 < BPW // NSLOT)
            def _(b1=b1, b2=b2, sem=sem, k=k):
                _fetch(t1_hbm, t2_hbm, c_s[k + NSLOT], x_s[k + NSLOT],
                       b1, b2, sem)

        @pl.when(g % 4 == 3)
        def _():
            y = 1.0 / (1.0 + jnp.exp(-acc))
            out_v[pl.ds((g // 4) * LANES, LANES)] = y

        return jnp.where(g % 4 == 3, jnp.zeros((LANES,), jnp.float32), acc)

    lax.fori_loop(0, BPW // NSLOT, ring_body,
                  jnp.zeros((LANES,), jnp.float32))

    pltpu.sync_copy(out_v, out_hbm.at[pl.ds(base, BPW)])


def kernel(center, context, table1, table2):
    mesh = plsc.VectorSubcoreMesh(core_axis_name="c", subcore_axis_name="s",
                                  num_cores=NC, num_subcores=NS)
    run = pl.kernel(
        _sc_body,
        out_type=jax.ShapeDtypeStruct((BATCH,), jnp.float32),
        mesh=mesh,
        compiler_params=pltpu.CompilerParams(needs_layout_passes=False,
                                             use_tc_tiling_on_sc=True),
        scratch_types=[
            pltpu.VMEM((NICHUNK, ICHUNK), jnp.int32),
            pltpu.VMEM((NICHUNK, ICHUNK), jnp.int32),
            pltpu.SMEM((BPW,), jnp.int32),
            pltpu.SMEM((BPW,), jnp.int32),
            pltpu.VMEM((EMB_DIM, BLK), jnp.float32),
            pltpu.VMEM((EMB_DIM, BLK), jnp.float32),
            pltpu.VMEM((EMB_DIM, BLK), jnp.float32),
            pltpu.VMEM((EMB_DIM, BLK), jnp.float32),
            pltpu.VMEM((EMB_DIM, BLK), jnp.float32),
            pltpu.VMEM((EMB_DIM, BLK), jnp.float32),
            pltpu.VMEM((EMB_DIM, BLK), jnp.float32),
            pltpu.VMEM((EMB_DIM, BLK), jnp.float32),
            pltpu.VMEM((BPW,), jnp.float32),
            pltpu.SemaphoreType.DMA,
            pltpu.SemaphoreType.DMA,
            pltpu.SemaphoreType.DMA,
            pltpu.SemaphoreType.DMA,
        ],
    )
    return run(center.astype(jnp.int32), context.astype(jnp.int32),
               jnp.swapaxes(table1, 0, 1), jnp.swapaxes(table2, 0, 1))


# speedup vs baseline: 2.9341x; 1.1151x over previous
import functools

import jax
import jax.numpy as jnp
from jax import lax
from jax.experimental import pallas as pl
from jax.experimental.pallas import tpu as pltpu
from jax.experimental.pallas import tpu_sc as plsc

NITEM = 1000000
EMB_DIM = 64
BATCH = 16384

NC = 2
NS = 16
LANES = 16
NW = NC * NS
BPW = BATCH // NW
BLK = 128
ICHUNK = 128
NICHUNK = BPW // ICHUNK


def _fetch(t1_hbm, t2_hbm, idx1, idx2, b1, b2, sem):
    base1 = pl.multiple_of((idx1 >> 7) * BLK, BLK)
    base2 = pl.multiple_of((idx2 >> 7) * BLK, BLK)
    pltpu.async_copy(t1_hbm.at[:, pl.ds(base1, BLK)], b1, sem)
    pltpu.async_copy(t2_hbm.at[:, pl.ds(base2, BLK)], b2, sem)


def _sc_body(center_hbm, context_hbm, t1_hbm, t2_hbm, out_hbm,
             cidx_v, xidx_v, c_s, x_s,
             blk0_1, blk0_2, blk1_1, blk1_2, blk2_1, blk2_2, blk3_1, blk3_2,
             blk4_1, blk4_2, blk5_1, blk5_2,
             out_v, sem0, sem1, sem2, sem3, sem4, sem5):
    wid = lax.axis_index("s") * NC + lax.axis_index("c")
    base = wid * BPW

    for j in range(NICHUNK):
        pltpu.sync_copy(center_hbm.at[pl.ds(base + j * ICHUNK, ICHUNK)],
                        cidx_v.at[j])
        pltpu.sync_copy(context_hbm.at[pl.ds(base + j * ICHUNK, ICHUNK)],
                        xidx_v.at[j])

    def spill_body(g, carry):
        j = g // (ICHUNK // LANES)
        gg = g % (ICHUNK // LANES)
        v1 = cidx_v[j, pl.ds(gg * LANES, LANES)]
        v2 = xidx_v[j, pl.ds(gg * LANES, LANES)]
        for r in range(LANES):
            c_s[g * LANES + r] = v1[r]
            x_s[g * LANES + r] = v2[r]
        return carry

    lax.fori_loop(0, BPW // LANES, spill_body, 0)

    iota = lax.iota(jnp.int32, LANES)
    c_vecs = [kc * LANES + iota for kc in range(EMB_DIM // LANES)]

    slots = [(blk0_1, blk0_2, sem0), (blk1_1, blk1_2, sem1),
             (blk2_1, blk2_2, sem2), (blk3_1, blk3_2, sem3),
             (blk4_1, blk4_2, sem4), (blk5_1, blk5_2, sem5)]
    NSLOT = len(slots)
    NTURN = (BPW + NSLOT - 1) // NSLOT

    for s, (b1, b2, sem) in enumerate(slots):
        _fetch(t1_hbm, t2_hbm, c_s[s], x_s[s], b1, b2, sem)

    def _dot(b1, b2, col1, col2, acc, lane):
        col1v = jnp.full((LANES,), col1, jnp.int32)
        col2v = jnp.full((LANES,), col2, jnp.int32)
        s = jnp.zeros((LANES,), jnp.float32)
        for kc in range(EMB_DIM // LANES):
            a = plsc.load_gather(b1, [c_vecs[kc], col1v])
            b = plsc.load_gather(b2, [c_vecs[kc], col2v])
            s = s + a * b
        return jnp.where(lane, jnp.sum(s), acc)

    def ring_body(g, acc):
        for s, (b1, b2, sem) in enumerate(slots):
            k = NSLOT * g + s

            def _consume(acc, b1=b1, b2=b2, sem=sem, k=k):
                pltpu.make_async_copy(t1_hbm.at[:, pl.ds(0, BLK)],
                                      b1, sem).wait()
                pltpu.make_async_copy(t1_hbm.at[:, pl.ds(0, BLK)],
                                      b2, sem).wait()
                acc = _dot(b1, b2, c_s[k] & (BLK - 1), x_s[k] & (BLK - 1),
                           acc, iota == (k % LANES))

                @pl.when(k + NSLOT < BPW)
                def _():
                    _fetch(t1_hbm, t2_hbm, c_s[k + NSLOT], x_s[k + NSLOT],
                           b1, b2, sem)

                @pl.when(k % LANES == LANES - 1)
                def _():
                    y = 1.0 / (1.0 + jnp.exp(-acc))
                    out_v[pl.ds((k // LANES) * LANES, LANES)] = y

                return jnp.where(k % LANES == LANES - 1,
                                 jnp.zeros((LANES,), jnp.float32), acc)

            if BPW % NSLOT == 0:
                acc = _consume(acc)
            else:
                acc = jax.lax.cond(k < BPW, _consume, lambda a: a, acc)
        return acc

    lax.fori_loop(0, NTURN, ring_body, jnp.zeros((LANES,), jnp.float32))

    pltpu.sync_copy(out_v, out_hbm.at[pl.ds(base, BPW)])


def kernel(center, context, table1, table2):
    mesh = plsc.VectorSubcoreMesh(core_axis_name="c", subcore_axis_name="s",
                                  num_cores=NC, num_subcores=NS)
    run = pl.kernel(
        _sc_body,
        out_type=jax.ShapeDtypeStruct((BATCH,), jnp.float32),
        mesh=mesh,
        compiler_params=pltpu.CompilerParams(needs_layout_passes=False,
                                             use_tc_tiling_on_sc=True),
        scratch_types=[
            pltpu.VMEM((NICHUNK, ICHUNK), jnp.int32),
            pltpu.VMEM((NICHUNK, ICHUNK), jnp.int32),
            pltpu.SMEM((BPW,), jnp.int32),
            pltpu.SMEM((BPW,), jnp.int32),
            pltpu.VMEM((EMB_DIM, BLK), jnp.float32),
            pltpu.VMEM((EMB_DIM, BLK), jnp.float32),
            pltpu.VMEM((EMB_DIM, BLK), jnp.float32),
            pltpu.VMEM((EMB_DIM, BLK), jnp.float32),
            pltpu.VMEM((EMB_DIM, BLK), jnp.float32),
            pltpu.VMEM((EMB_DIM, BLK), jnp.float32),
            pltpu.VMEM((EMB_DIM, BLK), jnp.float32),
            pltpu.VMEM((EMB_DIM, BLK), jnp.float32),
            pltpu.VMEM((EMB_DIM, BLK), jnp.float32),
            pltpu.VMEM((EMB_DIM, BLK), jnp.float32),
            pltpu.VMEM((EMB_DIM, BLK), jnp.float32),
            pltpu.VMEM((EMB_DIM, BLK), jnp.float32),
            pltpu.VMEM((BPW,), jnp.float32),
            pltpu.SemaphoreType.DMA,
            pltpu.SemaphoreType.DMA,
            pltpu.SemaphoreType.DMA,
            pltpu.SemaphoreType.DMA,
            pltpu.SemaphoreType.DMA,
            pltpu.SemaphoreType.DMA,
        ],
    )
    return run(center.astype(jnp.int32), context.astype(jnp.int32),
               jnp.swapaxes(table1, 0, 1), jnp.swapaxes(table2, 0, 1))


# speedup vs baseline: 2.9522x; 1.0062x over previous
import functools

import jax
import jax.numpy as jnp
from jax import lax
from jax.experimental import pallas as pl
from jax.experimental.pallas import tpu as pltpu
from jax.experimental.pallas import tpu_sc as plsc

NITEM = 1000000
EMB_DIM = 64
BATCH = 16384

NC = 2
NS = 16
LANES = 16
NW = NC * NS
BPW = BATCH // NW
BLK = 128
ICHUNK = 128
NICHUNK = BPW // ICHUNK


def _fetch(t1_hbm, t2_hbm, idx1, idx2, b1, b2, sem):
    base1 = pl.multiple_of((idx1 >> 7) * BLK, BLK)
    base2 = pl.multiple_of((idx2 >> 7) * BLK, BLK)
    pltpu.async_copy(t1_hbm.at[:, pl.ds(base1, BLK)], b1, sem)
    pltpu.async_copy(t2_hbm.at[:, pl.ds(base2, BLK)], b2, sem)


def _sc_body(center_hbm, context_hbm, t1_hbm, t2_hbm, out_hbm,
             cidx_v, xidx_v, c_s, x_s,
             blk0_1, blk0_2, blk1_1, blk1_2, blk2_1, blk2_2, blk3_1, blk3_2,
             blk4_1, blk4_2, blk5_1, blk5_2, blk6_1, blk6_2,
             out_v, sem0, sem1, sem2, sem3, sem4, sem5, sem6):
    wid = lax.axis_index("s") * NC + lax.axis_index("c")
    base = wid * BPW

    for j in range(NICHUNK):
        pltpu.sync_copy(center_hbm.at[pl.ds(base + j * ICHUNK, ICHUNK)],
                        cidx_v.at[j])
        pltpu.sync_copy(context_hbm.at[pl.ds(base + j * ICHUNK, ICHUNK)],
                        xidx_v.at[j])

    def spill_body(g, carry):
        j = g // (ICHUNK // LANES)
        gg = g % (ICHUNK // LANES)
        v1 = cidx_v[j, pl.ds(gg * LANES, LANES)]
        v2 = xidx_v[j, pl.ds(gg * LANES, LANES)]
        for r in range(LANES):
            c_s[g * LANES + r] = v1[r]
            x_s[g * LANES + r] = v2[r]
        return carry

    lax.fori_loop(0, BPW // LANES, spill_body, 0)

    iota = lax.iota(jnp.int32, LANES)
    c_vecs = [kc * LANES + iota for kc in range(EMB_DIM // LANES)]

    slots = [(blk0_1, blk0_2, sem0), (blk1_1, blk1_2, sem1),
             (blk2_1, blk2_2, sem2), (blk3_1, blk3_2, sem3),
             (blk4_1, blk4_2, sem4), (blk5_1, blk5_2, sem5),
             (blk6_1, blk6_2, sem6)]
    NSLOT = len(slots)
    NTURN = (BPW + NSLOT - 1) // NSLOT

    for s, (b1, b2, sem) in enumerate(slots):
        _fetch(t1_hbm, t2_hbm, c_s[s], x_s[s], b1, b2, sem)

    def _dot(b1, b2, col1, col2, acc, lane):
        col1v = jnp.full((LANES,), col1, jnp.int32)
        col2v = jnp.full((LANES,), col2, jnp.int32)
        s = jnp.zeros((LANES,), jnp.float32)
        for kc in range(EMB_DIM // LANES):
            a = plsc.load_gather(b1, [c_vecs[kc], col1v])
            b = plsc.load_gather(b2, [c_vecs[kc], col2v])
            s = s + a * b
        return jnp.where(lane, jnp.sum(s), acc)

    def ring_body(g, acc):
        for s, (b1, b2, sem) in enumerate(slots):
            k = NSLOT * g + s

            def _consume(acc, b1=b1, b2=b2, sem=sem, k=k):
                pltpu.make_async_copy(t1_hbm.at[:, pl.ds(0, BLK)],
                                      b1, sem).wait()
                pltpu.make_async_copy(t1_hbm.at[:, pl.ds(0, BLK)],
                                      b2, sem).wait()
                acc = _dot(b1, b2, c_s[k] & (BLK - 1), x_s[k] & (BLK - 1),
                           acc, iota == (k % LANES))

                @pl.when(k + NSLOT < BPW)
                def _():
                    _fetch(t1_hbm, t2_hbm, c_s[k + NSLOT], x_s[k + NSLOT],
                           b1, b2, sem)

                @pl.when(k % LANES == LANES - 1)
                def _():
                    y = 1.0 / (1.0 + jnp.exp(-acc))
                    out_v[pl.ds((k // LANES) * LANES, LANES)] = y

                return jnp.where(k % LANES == LANES - 1,
                                 jnp.zeros((LANES,), jnp.float32), acc)

            if BPW % NSLOT == 0:
                acc = _consume(acc)
            else:
                acc = jax.lax.cond(k < BPW, _consume, lambda a: a, acc)
        return acc

    lax.fori_loop(0, NTURN, ring_body, jnp.zeros((LANES,), jnp.float32))

    pltpu.sync_copy(out_v, out_hbm.at[pl.ds(base, BPW)])


def kernel(center, context, table1, table2):
    mesh = plsc.VectorSubcoreMesh(core_axis_name="c", subcore_axis_name="s",
                                  num_cores=NC, num_subcores=NS)
    run = pl.kernel(
        _sc_body,
        out_type=jax.ShapeDtypeStruct((BATCH,), jnp.float32),
        mesh=mesh,
        compiler_params=pltpu.CompilerParams(needs_layout_passes=False,
                                             use_tc_tiling_on_sc=True),
        scratch_types=[
            pltpu.VMEM((NICHUNK, ICHUNK), jnp.int32),
            pltpu.VMEM((NICHUNK, ICHUNK), jnp.int32),
            pltpu.SMEM((BPW,), jnp.int32),
            pltpu.SMEM((BPW,), jnp.int32),
            pltpu.VMEM((EMB_DIM, BLK), jnp.float32),
            pltpu.VMEM((EMB_DIM, BLK), jnp.float32),
            pltpu.VMEM((EMB_DIM, BLK), jnp.float32),
            pltpu.VMEM((EMB_DIM, BLK), jnp.float32),
            pltpu.VMEM((EMB_DIM, BLK), jnp.float32),
            pltpu.VMEM((EMB_DIM, BLK), jnp.float32),
            pltpu.VMEM((EMB_DIM, BLK), jnp.float32),
            pltpu.VMEM((EMB_DIM, BLK), jnp.float32),
            pltpu.VMEM((EMB_DIM, BLK), jnp.float32),
            pltpu.VMEM((EMB_DIM, BLK), jnp.float32),
            pltpu.VMEM((EMB_DIM, BLK), jnp.float32),
            pltpu.VMEM((EMB_DIM, BLK), jnp.float32),
            pltpu.VMEM((EMB_DIM, BLK), jnp.float32),
            pltpu.VMEM((EMB_DIM, BLK), jnp.float32),
            pltpu.VMEM((BPW,), jnp.float32),
            pltpu.SemaphoreType.DMA,
            pltpu.SemaphoreType.DMA,
            pltpu.SemaphoreType.DMA,
            pltpu.SemaphoreType.DMA,
            pltpu.SemaphoreType.DMA,
            pltpu.SemaphoreType.DMA,
            pltpu.SemaphoreType.DMA,
        ],
    )
    return run(center.astype(jnp.int32), context.astype(jnp.int32),
               jnp.swapaxes(table1, 0, 1), jnp.swapaxes(table2, 0, 1))
